# Initial kernel scaffold; baseline (speedup 1.0000x reference)
import jax
import jax.numpy as jnp
from jax.experimental import pallas as pl


def kernel(x, edge_index, W1, b1, W2, b2, Wl1, bl1, Wr1, Wl2, bl2, Wr2, Wl3, bl3, Wr3, W3, b3, W4, b4):
    raise NotImplementedError("write your pallas kernel here")


# speedup vs baseline: 16.0377x; 16.0377x over previous
import functools

import jax
import jax.numpy as jnp
from jax import lax
from jax.experimental import pallas as pl
from jax.experimental.pallas import tpu as pltpu
from jax.experimental.pallas import tpu_sc as plsc

N = 100000
E = 3200000
PADW = 16
NC = 2
NS = 16
SUB = 80
NSUB = 10
CHUNK = SUB * NSUB
EPW = E // NS
NIT = EPW // CHUNK
RPW = N // NS
ZCH = 250
HIGH = jax.lax.Precision.HIGHEST


def _halves(h, blk):
    ones = jnp.ones((blk, 1), jnp.float32)
    z3 = jnp.zeros((blk, 3), jnp.float32)
    z4 = jnp.zeros((blk, 4), jnp.float32)
    a = jnp.concatenate([h[:, :12], ones, z3], axis=1)
    b = jnp.concatenate([h[:, 12:], z4], axis=1)
    return a, b


def _enc_body(x_ref, w1_ref, b1_ref, w2_ref, b2_ref, outa_ref, outb_ref):
    h = jnp.maximum(jnp.dot(x_ref[...], w1_ref[...], precision=HIGH) + b1_ref[...], 0.0)
    h = jnp.maximum(jnp.dot(h, w2_ref[...], precision=HIGH) + b2_ref[...], 0.0)
    a, b = _halves(h, h.shape[0])
    outa_ref[...] = a
    outb_ref[...] = b


def _encoder(x, W1, b1, W2, b2, blk=2000):
    return pl.pallas_call(
        _enc_body,
        grid=(N // blk,),
        in_specs=[
            pl.BlockSpec((blk, 128), lambda i: (i, 0)),
            pl.BlockSpec((128, 240), lambda i: (0, 0)),
            pl.BlockSpec((1, 240), lambda i: (0, 0)),
            pl.BlockSpec((240, 24), lambda i: (0, 0)),
            pl.BlockSpec((1, 24), lambda i: (0, 0)),
        ],
        out_specs=[
            pl.BlockSpec((blk, PADW), lambda i: (i, 0)),
            pl.BlockSpec((blk, PADW), lambda i: (i, 0)),
        ],
        out_shape=[jax.ShapeDtypeStruct((N, PADW), jnp.float32)] * 2,
    )(x, W1, b1.reshape(1, 240), W2, b2.reshape(1, 24))


def _agg_body(src3, dst3, ha, hb, agga, aggb,
              src_v, dst_v, rows_v, zbuf, acc, sem):
    c = lax.axis_index("c")
    s = lax.axis_index("s")

    def _zrow(i, carry):
        zbuf[i] = jnp.zeros((PADW,), jnp.float32)
        return carry
    lax.fori_loop(0, ZCH, _zrow, 0)

    def _zcp(j, carry):
        pltpu.sync_copy(zbuf, acc.at[pl.ds(s * RPW + j * ZCH, ZCH)])
        return carry
    lax.fori_loop(0, RPW // ZCH, _zcp, 0)
    plsc.subcore_barrier()

    def _chunk(g, carry):
        ck = s * NIT + g
        pltpu.sync_copy(src3.at[ck], src_v)
        pltpu.sync_copy(dst3.at[ck], dst_v)

        @pl.when(c == 0)
        def _():
            descs = [pltpu.async_copy(ha.at[src_v.at[j]], rows_v.at[j], sem)
                     for j in range(NSUB)]
            for d in descs:
                d.wait()

        @pl.when(c == 1)
        def _():
            descs = [pltpu.async_copy(hb.at[src_v.at[j]], rows_v.at[j], sem)
                     for j in range(NSUB)]
            for d in descs:
                d.wait()

        for j in range(NSUB):
            pltpu.sync_copy(rows_v.at[j], acc.at[dst_v.at[j]], add=True)
        return carry
    lax.fori_loop(0, NIT, _chunk, 0)
    plsc.subcore_barrier()

    @pl.when(c == 0)
    def _():
        pltpu.sync_copy(acc.at[pl.ds(s * RPW, RPW)], agga.at[pl.ds(s * RPW, RPW)])

    @pl.when(c == 1)
    def _():
        pltpu.sync_copy(acc.at[pl.ds(s * RPW, RPW)], aggb.at[pl.ds(s * RPW, RPW)])


_agg = functools.partial(
    pl.kernel,
    out_type=[jax.ShapeDtypeStruct((N, PADW), jnp.float32)] * 2,
    mesh=plsc.VectorSubcoreMesh(core_axis_name="c", subcore_axis_name="s"),
    scratch_types=[
        pltpu.VMEM((NSUB, SUB), jnp.int32),
        pltpu.VMEM((NSUB, SUB), jnp.int32),
        pltpu.VMEM((NSUB, SUB, PADW), jnp.float32),
        pltpu.VMEM((ZCH, PADW), jnp.float32),
        pltpu.VMEM_SHARED((N, PADW), jnp.float32),
        pltpu.SemaphoreType.DMA,
    ],
    compiler_params=pltpu.CompilerParams(use_tc_tiling_on_sc=False),
)(_agg_body)


def _mean_and_h(agga, aggb, ha, hb):
    cnt = jnp.maximum(agga[:, 12:13], 1.0)
    mean = jnp.concatenate([agga[:, :12], aggb[:, :12]], axis=1) / cnt
    h = jnp.concatenate([ha[:, :12], hb[:, :12]], axis=1)
    return mean, h


def _sage_body(agga_ref, aggb_ref, ha_ref, hb_ref, wl_ref, bl_ref, wr_ref,
               outa_ref, outb_ref):
    mean, h = _mean_and_h(agga_ref[...], aggb_ref[...], ha_ref[...], hb_ref[...])
    o = (jnp.dot(mean, wl_ref[...], precision=HIGH) + bl_ref[...]
         + jnp.dot(h, wr_ref[...], precision=HIGH))
    o = jnp.maximum(o, 0.0)
    a, b = _halves(o, o.shape[0])
    outa_ref[...] = a
    outb_ref[...] = b


def _sage(agga, aggb, ha, hb, Wl, bl, Wr, blk=2000):
    return pl.pallas_call(
        _sage_body,
        grid=(N // blk,),
        in_specs=[
            pl.BlockSpec((blk, PADW), lambda i: (i, 0)),
            pl.BlockSpec((blk, PADW), lambda i: (i, 0)),
            pl.BlockSpec((blk, PADW), lambda i: (i, 0)),
            pl.BlockSpec((blk, PADW), lambda i: (i, 0)),
            pl.BlockSpec((24, 24), lambda i: (0, 0)),
            pl.BlockSpec((1, 24), lambda i: (0, 0)),
            pl.BlockSpec((24, 24), lambda i: (0, 0)),
        ],
        out_specs=[
            pl.BlockSpec((blk, PADW), lambda i: (i, 0)),
            pl.BlockSpec((blk, PADW), lambda i: (i, 0)),
        ],
        out_shape=[jax.ShapeDtypeStruct((N, PADW), jnp.float32)] * 2,
    )(agga, aggb, ha, hb, Wl, bl.reshape(1, 24), Wr)


def _sage3_dec_body(agga_ref, aggb_ref, ha_ref, hb_ref, wl_ref, bl_ref, wr_ref,
                    w3_ref, b3_ref, w4_ref, b4_ref, out_ref):
    mean, h = _mean_and_h(agga_ref[...], aggb_ref[...], ha_ref[...], hb_ref[...])
    o = (jnp.dot(mean, wl_ref[...], precision=HIGH) + bl_ref[...]
         + jnp.dot(h, wr_ref[...], precision=HIGH))
    d = jnp.maximum(jnp.dot(o, w3_ref[...], precision=HIGH) + b3_ref[...], 0.0)
    out_ref[...] = jnp.dot(d, w4_ref[...], precision=HIGH) + b4_ref[...]


def _sage3_dec(agga, aggb, ha, hb, Wl, bl, Wr, W3, b3, W4, b4, blk=2000):
    return pl.pallas_call(
        _sage3_dec_body,
        grid=(N // blk,),
        in_specs=[
            pl.BlockSpec((blk, PADW), lambda i: (i, 0)),
            pl.BlockSpec((blk, PADW), lambda i: (i, 0)),
            pl.BlockSpec((blk, PADW), lambda i: (i, 0)),
            pl.BlockSpec((blk, PADW), lambda i: (i, 0)),
            pl.BlockSpec((24, 24), lambda i: (0, 0)),
            pl.BlockSpec((1, 24), lambda i: (0, 0)),
            pl.BlockSpec((24, 24), lambda i: (0, 0)),
            pl.BlockSpec((24, 120), lambda i: (0, 0)),
            pl.BlockSpec((1, 120), lambda i: (0, 0)),
            pl.BlockSpec((120, 12), lambda i: (0, 0)),
            pl.BlockSpec((1, 12), lambda i: (0, 0)),
        ],
        out_specs=pl.BlockSpec((blk, 12), lambda i: (i, 0)),
        out_shape=jax.ShapeDtypeStruct((N, 12), jnp.float32),
    )(agga, aggb, ha, hb, Wl, bl.reshape(1, 24), Wr,
      W3, b3.reshape(1, 120), W4, b4.reshape(1, 12))


def kernel(x, edge_index, W1, b1, W2, b2, Wl1, bl1, Wr1, Wl2, bl2, Wr2,
           Wl3, bl3, Wr3, W3, b3, W4, b4):
    src3 = edge_index[0].reshape(E // CHUNK, NSUB, SUB)
    dst3 = edge_index[1].reshape(E // CHUNK, NSUB, SUB)

    ha, hb = _encoder(x, W1, b1, W2, b2)
    agga, aggb = _agg(src3, dst3, ha, hb)
    ha, hb = _sage(agga, aggb, ha, hb, Wl1, bl1, Wr1)
    agga, aggb = _agg(src3, dst3, ha, hb)
    ha, hb = _sage(agga, aggb, ha, hb, Wl2, bl2, Wr2)
    agga, aggb = _agg(src3, dst3, ha, hb)
    return _sage3_dec(agga, aggb, ha, hb, Wl3, bl3, Wr3, W3, b3, W4, b4)


# speedup vs baseline: 24.6432x; 1.5366x over previous
import functools

import jax
import jax.numpy as jnp
from jax import lax
from jax.experimental import pallas as pl
from jax.experimental.pallas import tpu as pltpu
from jax.experimental.pallas import tpu_sc as plsc

N = 100000
E = 3200000
PADW = 16
NC = 2
NS = 16
SUB = 80
NSUB = 10
CHUNK = SUB * NSUB
EPW = E // NS
NIT = EPW // CHUNK
RPW = N // NS
ZCH = 250
HIGH = jax.lax.Precision.HIGHEST


def _halves(h, blk):
    ones = jnp.ones((blk, 1), jnp.float32)
    z3 = jnp.zeros((blk, 3), jnp.float32)
    z4 = jnp.zeros((blk, 4), jnp.float32)
    a = jnp.concatenate([h[:, :12], ones, z3], axis=1)
    b = jnp.concatenate([h[:, 12:], z4], axis=1)
    return a, b


def _enc_body(x_ref, w1_ref, b1_ref, w2_ref, b2_ref, outa_ref, outb_ref):
    h = jnp.maximum(jnp.dot(x_ref[...], w1_ref[...], precision=HIGH) + b1_ref[...], 0.0)
    h = jnp.maximum(jnp.dot(h, w2_ref[...], precision=HIGH) + b2_ref[...], 0.0)
    a, b = _halves(h, h.shape[0])
    outa_ref[...] = a
    outb_ref[...] = b


def _encoder(x, W1, b1, W2, b2, blk=2000):
    return pl.pallas_call(
        _enc_body,
        grid=(N // blk,),
        in_specs=[
            pl.BlockSpec((blk, 128), lambda i: (i, 0)),
            pl.BlockSpec((128, 240), lambda i: (0, 0)),
            pl.BlockSpec((1, 240), lambda i: (0, 0)),
            pl.BlockSpec((240, 24), lambda i: (0, 0)),
            pl.BlockSpec((1, 24), lambda i: (0, 0)),
        ],
        out_specs=[
            pl.BlockSpec((blk, PADW), lambda i: (i, 0)),
            pl.BlockSpec((blk, PADW), lambda i: (i, 0)),
        ],
        out_shape=[jax.ShapeDtypeStruct((N, PADW), jnp.float32)] * 2,
    )(x, W1, b1.reshape(1, 240), W2, b2.reshape(1, 24))


def _agg_body(src3, dst3, ha, hb, zeros, agga, aggb,
              src_v0, dst_v0, rows_v0, src_v1, dst_v1, rows_v1,
              acc, semg0, semg1, sems0, sems1):
    c = lax.axis_index("c")
    s = lax.axis_index("s")
    bufs = ((src_v0, dst_v0, rows_v0, semg0, sems0),
            (src_v1, dst_v1, rows_v1, semg1, sems1))

    pltpu.sync_copy(zeros.at[pl.ds(s * RPW, RPW)], acc.at[pl.ds(s * RPW, RPW)])
    plsc.subcore_barrier()

    def fire(g, buf):
        src_v, dst_v, rows_v, semg, _ = buf
        ck = s * NIT + g
        pltpu.sync_copy(src3.at[ck], src_v)
        pltpu.sync_copy(dst3.at[ck], dst_v)

        @pl.when(c == 0)
        def _():
            for j in range(NSUB):
                pltpu.async_copy(ha.at[src_v.at[j]], rows_v.at[j], semg)

        @pl.when(c == 1)
        def _():
            for j in range(NSUB):
                pltpu.async_copy(hb.at[src_v.at[j]], rows_v.at[j], semg)

    def wait_gather(buf):
        src_v, _, rows_v, semg, _ = buf
        for j in range(NSUB):
            pltpu.make_async_copy(ha.at[src_v.at[j]], rows_v.at[j], semg).wait()

    def fire_scatter(buf):
        _, dst_v, rows_v, _, sems = buf
        for j in range(NSUB):
            pltpu.async_copy(rows_v.at[j], acc.at[dst_v.at[j]], sems, add=True)

    def wait_scatter(buf):
        _, dst_v, rows_v, _, sems = buf
        for j in range(NSUB):
            pltpu.make_async_copy(rows_v.at[j], acc.at[dst_v.at[j]], sems).wait()

    fire(0, bufs[0])

    def _pair(i, carry):
        g0 = 2 * i

        @pl.when(i > 0)
        def _():
            wait_scatter(bufs[1])
        fire(g0 + 1, bufs[1])
        wait_gather(bufs[0])
        fire_scatter(bufs[0])

        wait_scatter(bufs[0])

        @pl.when(g0 + 2 < NIT)
        def _():
            fire(g0 + 2, bufs[0])
        wait_gather(bufs[1])
        fire_scatter(bufs[1])
        return carry
    lax.fori_loop(0, NIT // 2, _pair, 0)
    wait_scatter(bufs[1])
    plsc.subcore_barrier()

    @pl.when(c == 0)
    def _():
        pltpu.sync_copy(acc.at[pl.ds(s * RPW, RPW)], agga.at[pl.ds(s * RPW, RPW)])

    @pl.when(c == 1)
    def _():
        pltpu.sync_copy(acc.at[pl.ds(s * RPW, RPW)], aggb.at[pl.ds(s * RPW, RPW)])


_agg = functools.partial(
    pl.kernel,
    out_type=[jax.ShapeDtypeStruct((N, PADW), jnp.float32)] * 2,
    mesh=plsc.VectorSubcoreMesh(core_axis_name="c", subcore_axis_name="s"),
    scratch_types=[
        pltpu.VMEM((NSUB, SUB), jnp.int32),
        pltpu.VMEM((NSUB, SUB), jnp.int32),
        pltpu.VMEM((NSUB, SUB, PADW), jnp.float32),
        pltpu.VMEM((NSUB, SUB), jnp.int32),
        pltpu.VMEM((NSUB, SUB), jnp.int32),
        pltpu.VMEM((NSUB, SUB, PADW), jnp.float32),
        pltpu.VMEM_SHARED((N, PADW), jnp.float32),
        pltpu.SemaphoreType.DMA,
        pltpu.SemaphoreType.DMA,
        pltpu.SemaphoreType.DMA,
        pltpu.SemaphoreType.DMA,
    ],
    compiler_params=pltpu.CompilerParams(use_tc_tiling_on_sc=False),
)(_agg_body)


def _mean_and_h(agga, aggb, ha, hb):
    cnt = jnp.maximum(agga[:, 12:13], 1.0)
    mean = jnp.concatenate([agga[:, :12], aggb[:, :12]], axis=1) / cnt
    h = jnp.concatenate([ha[:, :12], hb[:, :12]], axis=1)
    return mean, h


def _sage_body(agga_ref, aggb_ref, ha_ref, hb_ref, wl_ref, bl_ref, wr_ref,
               outa_ref, outb_ref):
    mean, h = _mean_and_h(agga_ref[...], aggb_ref[...], ha_ref[...], hb_ref[...])
    o = (jnp.dot(mean, wl_ref[...], precision=HIGH) + bl_ref[...]
         + jnp.dot(h, wr_ref[...], precision=HIGH))
    o = jnp.maximum(o, 0.0)
    a, b = _halves(o, o.shape[0])
    outa_ref[...] = a
    outb_ref[...] = b


def _sage(agga, aggb, ha, hb, Wl, bl, Wr, blk=2000):
    return pl.pallas_call(
        _sage_body,
        grid=(N // blk,),
        in_specs=[
            pl.BlockSpec((blk, PADW), lambda i: (i, 0)),
            pl.BlockSpec((blk, PADW), lambda i: (i, 0)),
            pl.BlockSpec((blk, PADW), lambda i: (i, 0)),
            pl.BlockSpec((blk, PADW), lambda i: (i, 0)),
            pl.BlockSpec((24, 24), lambda i: (0, 0)),
            pl.BlockSpec((1, 24), lambda i: (0, 0)),
            pl.BlockSpec((24, 24), lambda i: (0, 0)),
        ],
        out_specs=[
            pl.BlockSpec((blk, PADW), lambda i: (i, 0)),
            pl.BlockSpec((blk, PADW), lambda i: (i, 0)),
        ],
        out_shape=[jax.ShapeDtypeStruct((N, PADW), jnp.float32)] * 2,
    )(agga, aggb, ha, hb, Wl, bl.reshape(1, 24), Wr)


def _sage3_dec_body(agga_ref, aggb_ref, ha_ref, hb_ref, wl_ref, bl_ref, wr_ref,
                    w3_ref, b3_ref, w4_ref, b4_ref, out_ref):
    mean, h = _mean_and_h(agga_ref[...], aggb_ref[...], ha_ref[...], hb_ref[...])
    o = (jnp.dot(mean, wl_ref[...], precision=HIGH) + bl_ref[...]
         + jnp.dot(h, wr_ref[...], precision=HIGH))
    d = jnp.maximum(jnp.dot(o, w3_ref[...], precision=HIGH) + b3_ref[...], 0.0)
    out_ref[...] = jnp.dot(d, w4_ref[...], precision=HIGH) + b4_ref[...]


def _sage3_dec(agga, aggb, ha, hb, Wl, bl, Wr, W3, b3, W4, b4, blk=2000):
    return pl.pallas_call(
        _sage3_dec_body,
        grid=(N // blk,),
        in_specs=[
            pl.BlockSpec((blk, PADW), lambda i: (i, 0)),
            pl.BlockSpec((blk, PADW), lambda i: (i, 0)),
            pl.BlockSpec((blk, PADW), lambda i: (i, 0)),
            pl.BlockSpec((blk, PADW), lambda i: (i, 0)),
            pl.BlockSpec((24, 24), lambda i: (0, 0)),
            pl.BlockSpec((1, 24), lambda i: (0, 0)),
            pl.BlockSpec((24, 24), lambda i: (0, 0)),
            pl.BlockSpec((24, 120), lambda i: (0, 0)),
            pl.BlockSpec((1, 120), lambda i: (0, 0)),
            pl.BlockSpec((120, 12), lambda i: (0, 0)),
            pl.BlockSpec((1, 12), lambda i: (0, 0)),
        ],
        out_specs=pl.BlockSpec((blk, 12), lambda i: (i, 0)),
        out_shape=jax.ShapeDtypeStruct((N, 12), jnp.float32),
    )(agga, aggb, ha, hb, Wl, bl.reshape(1, 24), Wr,
      W3, b3.reshape(1, 120), W4, b4.reshape(1, 12))


def kernel(x, edge_index, W1, b1, W2, b2, Wl1, bl1, Wr1, Wl2, bl2, Wr2,
           Wl3, bl3, Wr3, W3, b3, W4, b4):
    src3 = edge_index[0].reshape(E // CHUNK, NSUB, SUB)
    dst3 = edge_index[1].reshape(E // CHUNK, NSUB, SUB)
    zeros = jnp.zeros((N, PADW), jnp.float32)

    ha, hb = _encoder(x, W1, b1, W2, b2)
    agga, aggb = _agg(src3, dst3, ha, hb, zeros)
    ha, hb = _sage(agga, aggb, ha, hb, Wl1, bl1, Wr1)
    agga, aggb = _agg(src3, dst3, ha, hb, zeros)
    ha, hb = _sage(agga, aggb, ha, hb, Wl2, bl2, Wr2)
    agga, aggb = _agg(src3, dst3, ha, hb, zeros)
    return _sage3_dec(agga, aggb, ha, hb, Wl3, bl3, Wr3, W3, b3, W4, b4)


# speedup vs baseline: 29.7672x; 1.2079x over previous
import functools

import jax
import jax.numpy as jnp
from jax import lax
from jax.experimental import pallas as pl
from jax.experimental.pallas import tpu as pltpu
from jax.experimental.pallas import tpu_sc as plsc

N = 100000
N2 = 102400
NP = N2 // 8
E = 3200000
PADW = 16
NC = 2
NS = 16
SUB = 80
NSUB = 10
CHUNK = SUB * NSUB
EPW = E // NS
NIT = EPW // CHUNK
RPW = N // NS
HIGH = jax.lax.Precision.HIGHEST


def _enc_body(x_ref, w1_ref, b1_ref, w2_ref, b2_ref, outa_ref, outb_ref):
    h = jnp.maximum(jnp.dot(x_ref[...], w1_ref[...], precision=HIGH) + b1_ref[...], 0.0)
    h = jnp.maximum(jnp.dot(h, w2_ref[...], precision=HIGH) + b2_ref[...], 0.0)
    blk = h.shape[0]
    ones = jnp.ones((blk, 1), jnp.float32)
    z3 = jnp.zeros((blk, 3), jnp.float32)
    z4 = jnp.zeros((blk, 4), jnp.float32)
    outa_ref[...] = jnp.concatenate([h[:, :12], ones, z3], axis=1)
    outb_ref[...] = jnp.concatenate([h[:, 12:], z4], axis=1)


def _encoder(x, W1, b1, W2, b2, blk=2000):
    return pl.pallas_call(
        _enc_body,
        grid=(N // blk,),
        in_specs=[
            pl.BlockSpec((blk, 128), lambda i: (i, 0)),
            pl.BlockSpec((128, 240), lambda i: (0, 0)),
            pl.BlockSpec((1, 240), lambda i: (0, 0)),
            pl.BlockSpec((240, 24), lambda i: (0, 0)),
            pl.BlockSpec((1, 24), lambda i: (0, 0)),
        ],
        out_specs=[
            pl.BlockSpec((blk, PADW), lambda i: (i, 0)),
            pl.BlockSpec((blk, PADW), lambda i: (i, 0)),
        ],
        out_shape=[jax.ShapeDtypeStruct((N2, PADW), jnp.float32)] * 2,
    )(x, W1, b1.reshape(1, 240), W2, b2.reshape(1, 24))


def _agg_body(src3, dst3, ha, hb, zeros, agga, aggb,
              src_v0, dst_v0, rows_v0, src_v1, dst_v1, rows_v1,
              acc, semg0, semg1, sems0, sems1):
    c = lax.axis_index("c")
    s = lax.axis_index("s")
    bufs = ((src_v0, dst_v0, rows_v0, semg0, sems0),
            (src_v1, dst_v1, rows_v1, semg1, sems1))

    pltpu.sync_copy(zeros.at[pl.ds(s * RPW, RPW)], acc.at[pl.ds(s * RPW, RPW)])
    plsc.subcore_barrier()

    def fire(g, buf):
        src_v, dst_v, rows_v, semg, _ = buf
        ck = s * NIT + g
        pltpu.sync_copy(src3.at[ck], src_v)
        pltpu.sync_copy(dst3.at[ck], dst_v)

        @pl.when(c == 0)
        def _():
            for j in range(NSUB):
                pltpu.async_copy(ha.at[src_v.at[j]], rows_v.at[j], semg)

        @pl.when(c == 1)
        def _():
            for j in range(NSUB):
                pltpu.async_copy(hb.at[src_v.at[j]], rows_v.at[j], semg)

    def wait_gather(buf):
        src_v, _, rows_v, semg, _ = buf
        for j in range(NSUB):
            pltpu.make_async_copy(ha.at[src_v.at[j]], rows_v.at[j], semg).wait()

    def fire_scatter(buf):
        _, dst_v, rows_v, _, sems = buf
        for j in range(NSUB):
            pltpu.async_copy(rows_v.at[j], acc.at[dst_v.at[j]], sems, add=True)

    def wait_scatter(buf):
        _, dst_v, rows_v, _, sems = buf
        for j in range(NSUB):
            pltpu.make_async_copy(rows_v.at[j], acc.at[dst_v.at[j]], sems).wait()

    fire(0, bufs[0])

    def _pair(i, carry):
        g0 = 2 * i

        @pl.when(i > 0)
        def _():
            wait_scatter(bufs[1])
        fire(g0 + 1, bufs[1])
        wait_gather(bufs[0])
        fire_scatter(bufs[0])

        wait_scatter(bufs[0])

        @pl.when(g0 + 2 < NIT)
        def _():
            fire(g0 + 2, bufs[0])
        wait_gather(bufs[1])
        fire_scatter(bufs[1])
        return carry
    lax.fori_loop(0, NIT // 2, _pair, 0)
    wait_scatter(bufs[1])
    plsc.subcore_barrier()

    @pl.when(c == 0)
    def _():
        pltpu.sync_copy(acc.at[pl.ds(s * RPW, RPW)], agga.at[pl.ds(s * RPW, RPW)])

    @pl.when(c == 1)
    def _():
        pltpu.sync_copy(acc.at[pl.ds(s * RPW, RPW)], aggb.at[pl.ds(s * RPW, RPW)])


_agg = functools.partial(
    pl.kernel,
    out_type=[jax.ShapeDtypeStruct((N2, PADW), jnp.float32)] * 2,
    mesh=plsc.VectorSubcoreMesh(core_axis_name="c", subcore_axis_name="s"),
    scratch_types=[
        pltpu.VMEM((NSUB, SUB), jnp.int32),
        pltpu.VMEM((NSUB, SUB), jnp.int32),
        pltpu.VMEM((NSUB, SUB, PADW), jnp.float32),
        pltpu.VMEM((NSUB, SUB), jnp.int32),
        pltpu.VMEM((NSUB, SUB), jnp.int32),
        pltpu.VMEM((NSUB, SUB, PADW), jnp.float32),
        pltpu.VMEM_SHARED((N, PADW), jnp.float32),
        pltpu.SemaphoreType.DMA,
        pltpu.SemaphoreType.DMA,
        pltpu.SemaphoreType.DMA,
        pltpu.SemaphoreType.DMA,
    ],
    compiler_params=pltpu.CompilerParams(use_tc_tiling_on_sc=False),
)(_agg_body)


def _bd(w12):
    w16 = jnp.zeros((16, 16), jnp.float32).at[:12, :12].set(w12)
    return jnp.kron(jnp.eye(8, dtype=jnp.float32), w16)


def _stacks(Wl, Wr):
    wa = jnp.concatenate([_bd(Wl[:12, :12]), _bd(Wl[12:, :12]),
                          _bd(Wr[:12, :12]), _bd(Wr[12:, :12])], axis=0)
    wb = jnp.concatenate([_bd(Wl[:12, 12:]), _bd(Wl[12:, 12:]),
                          _bd(Wr[:12, 12:]), _bd(Wr[12:, 12:])], axis=0)
    return wa, wb


def _spread():
    s16 = jnp.zeros((16, 16), jnp.float32).at[12, :].set(1.0)
    return jnp.kron(jnp.eye(8, dtype=jnp.float32), s16)


def _bias_tile(b12):
    return jnp.tile(jnp.concatenate([b12, jnp.zeros((4,), jnp.float32)]), 8).reshape(1, 128)


def _sagep_body(agga_ref, aggb_ref, ha_ref, hb_ref, wa_ref, wb_ref,
                ba_ref, bb_ref, sp_ref, outa_ref, outb_ref, *, final):
    agga = agga_ref[...]
    aggb = aggb_ref[...]
    cnt = jnp.dot(agga, sp_ref[...], precision=HIGH)
    inv = 1.0 / jnp.maximum(cnt, 1.0)
    xcat = jnp.concatenate([agga * inv, aggb * inv, ha_ref[...], hb_ref[...]], axis=1)
    oa = jnp.dot(xcat, wa_ref[...], precision=HIGH) + ba_ref[...]
    ob = jnp.dot(xcat, wb_ref[...], precision=HIGH) + bb_ref[...]
    if final:
        outa_ref[...] = oa
        outb_ref[...] = ob
    else:
        lane = lax.broadcasted_iota(jnp.int32, oa.shape, 1)
        onesa = jnp.where(lane % 16 == 12, 1.0, 0.0)
        outa_ref[...] = jnp.maximum(oa, 0.0) + onesa
        outb_ref[...] = jnp.maximum(ob, 0.0)


def _sagep(agga_p, aggb_p, ha_p, hb_p, Wl, bl, Wr, final=False, blk=1600):
    wa, wb = _stacks(Wl, Wr)
    return pl.pallas_call(
        functools.partial(_sagep_body, final=final),
        grid=(NP // blk,),
        in_specs=[
            pl.BlockSpec((blk, 128), lambda i: (i, 0)),
            pl.BlockSpec((blk, 128), lambda i: (i, 0)),
            pl.BlockSpec((blk, 128), lambda i: (i, 0)),
            pl.BlockSpec((blk, 128), lambda i: (i, 0)),
            pl.BlockSpec((512, 128), lambda i: (0, 0)),
            pl.BlockSpec((512, 128), lambda i: (0, 0)),
            pl.BlockSpec((1, 128), lambda i: (0, 0)),
            pl.BlockSpec((1, 128), lambda i: (0, 0)),
            pl.BlockSpec((128, 128), lambda i: (0, 0)),
        ],
        out_specs=[
            pl.BlockSpec((blk, 128), lambda i: (i, 0)),
            pl.BlockSpec((blk, 128), lambda i: (i, 0)),
        ],
        out_shape=[jax.ShapeDtypeStruct((NP, 128), jnp.float32)] * 2,
    )(agga_p, aggb_p, ha_p, hb_p, wa, wb,
      _bias_tile(bl[:12]), _bias_tile(bl[12:]), _spread())


def _dec_body(oa_ref, ob_ref, w3_ref, b3_ref, w4_ref, b4_ref, out_ref):
    o = jnp.concatenate([oa_ref[:, :12], ob_ref[:, :12]], axis=1)
    d = jnp.maximum(jnp.dot(o, w3_ref[...], precision=HIGH) + b3_ref[...], 0.0)
    out_ref[...] = jnp.dot(d, w4_ref[...], precision=HIGH) + b4_ref[...]


def _decoder(oa_t, ob_t, W3, b3, W4, b4, blk=3200):
    return pl.pallas_call(
        _dec_body,
        grid=(N2 // blk,),
        in_specs=[
            pl.BlockSpec((blk, PADW), lambda i: (i, 0)),
            pl.BlockSpec((blk, PADW), lambda i: (i, 0)),
            pl.BlockSpec((24, 120), lambda i: (0, 0)),
            pl.BlockSpec((1, 120), lambda i: (0, 0)),
            pl.BlockSpec((120, 12), lambda i: (0, 0)),
            pl.BlockSpec((1, 12), lambda i: (0, 0)),
        ],
        out_specs=pl.BlockSpec((blk, 12), lambda i: (i, 0)),
        out_shape=jax.ShapeDtypeStruct((N2, 12), jnp.float32),
    )(oa_t, ob_t, W3, b3.reshape(1, 120), W4, b4.reshape(1, 12))


def kernel(x, edge_index, W1, b1, W2, b2, Wl1, bl1, Wr1, Wl2, bl2, Wr2,
           Wl3, bl3, Wr3, W3, b3, W4, b4):
    src3 = edge_index[0].reshape(E // CHUNK, NSUB, SUB)
    dst3 = edge_index[1].reshape(E // CHUNK, NSUB, SUB)
    zeros = jnp.zeros((N, PADW), jnp.float32)

    def packed(t):
        return jnp.reshape(t, (NP, 128))

    def flat(p):
        return jnp.reshape(p, (N2, PADW))

    ha_t, hb_t = _encoder(x, W1, b1, W2, b2)
    ha_p, hb_p = packed(ha_t), packed(hb_t)

    agga, aggb = _agg(src3, dst3, flat(ha_p), flat(hb_p), zeros)
    ha_p, hb_p = _sagep(packed(agga), packed(aggb), ha_p, hb_p, Wl1, bl1, Wr1)
    agga, aggb = _agg(src3, dst3, flat(ha_p), flat(hb_p), zeros)
    ha_p, hb_p = _sagep(packed(agga), packed(aggb), ha_p, hb_p, Wl2, bl2, Wr2)
    agga, aggb = _agg(src3, dst3, flat(ha_p), flat(hb_p), zeros)
    oa_p, ob_p = _sagep(packed(agga), packed(aggb), ha_p, hb_p, Wl3, bl3, Wr3,
                        final=True)

    out = _decoder(flat(oa_p), flat(ob_p), W3, b3, W4, b4)
    return out[:N]
